# baseline (speedup 1.0000x reference)
import functools
import math

import jax
import jax.numpy as jnp
from jax import lax
from jax.experimental import pallas as pl
from jax.experimental.pallas import tpu as pltpu
from jax.experimental.pallas import tpu_sc as plsc

EMBED_DIM = 32
SCALE = math.sqrt(EMBED_DIM)

NUM_CORES = 2
NUM_SUBCORES = 16
NUM_WORKERS = NUM_CORES * NUM_SUBCORES

CHUNK = 512
SUB = 128
NSUB = CHUNK // SUB


@functools.lru_cache(maxsize=None)
def _build(batch: int):
    assert batch % (NUM_WORKERS * CHUNK) == 0
    rows_per_worker = batch // NUM_WORKERS
    num_chunks = rows_per_worker // CHUNK
    mesh = plsc.VectorSubcoreMesh(core_axis_name="c", subcore_axis_name="s")

    @functools.partial(
        pl.kernel,
        mesh=mesh,
        out_type=jax.ShapeDtypeStruct((batch, EMBED_DIM), jnp.float32),
        scratch_types=[
            pltpu.VMEM((CHUNK,), jnp.int32),
            pltpu.VMEM((CHUNK, EMBED_DIM), jnp.float32),
            pltpu.SemaphoreType.DMA,
        ],
        compiler_params=pltpu.CompilerParams(use_tc_tiling_on_sc=False),
    )
    def emb(idx_hbm, table_hbm, out_hbm, idx_v, rows_v, sem):
        wid = lax.axis_index("s") * NUM_CORES + lax.axis_index("c")
        base = wid * rows_per_worker

        def chunk_body(g, carry):
            off = base + g * CHUNK
            pltpu.sync_copy(idx_hbm.at[pl.ds(off, CHUNK)], idx_v)
            copies = []
            for j in range(NSUB):
                copies.append(
                    pltpu.async_copy(
                        table_hbm.at[idx_v.at[pl.ds(j * SUB, SUB)]],
                        rows_v.at[pl.ds(j * SUB, SUB)],
                        sem,
                    )
                )
            for c in copies:
                c.wait()

            def mul_body(r, carry2):
                rows_v[r, pl.ds(0, 16)] = rows_v[r, pl.ds(0, 16)] * SCALE
                rows_v[r, pl.ds(16, 16)] = rows_v[r, pl.ds(16, 16)] * SCALE
                return carry2

            lax.fori_loop(0, CHUNK, mul_body, 0)
            pltpu.sync_copy(rows_v, out_hbm.at[pl.ds(off, CHUNK)])
            return carry

        lax.fori_loop(0, num_chunks, chunk_body, 0)

    return emb


def kernel(x, table):
    idx = x.reshape(-1).astype(jnp.int32)
    out = _build(idx.shape[0])(idx, table)
    return out.reshape(x.shape + (EMBED_DIM,))


# speedup vs baseline: 1.1049x; 1.1049x over previous
import functools
import math

import jax
import jax.numpy as jnp
from jax import lax
from jax.experimental import pallas as pl
from jax.experimental.pallas import tpu as pltpu
from jax.experimental.pallas import tpu_sc as plsc

EMBED_DIM = 32
SCALE = math.sqrt(EMBED_DIM)

NUM_CORES = 2
NUM_SUBCORES = 16
NUM_WORKERS = NUM_CORES * NUM_SUBCORES

CHUNK = 640
SUB = 128
NSUB = CHUNK // SUB
NBUF = 4


@functools.lru_cache(maxsize=None)
def _build(batch: int):
    assert batch % (NUM_WORKERS * CHUNK * NBUF) == 0
    rows_per_worker = batch // NUM_WORKERS
    num_chunks = rows_per_worker // CHUNK
    num_super = num_chunks // NBUF
    mesh = plsc.VectorSubcoreMesh(core_axis_name="c", subcore_axis_name="s")

    @functools.partial(
        pl.kernel,
        mesh=mesh,
        out_type=jax.ShapeDtypeStruct((batch, EMBED_DIM), jnp.float32),
        scratch_types=[
            pltpu.VMEM((rows_per_worker,), jnp.int32),
            pltpu.VMEM((NBUF, CHUNK, EMBED_DIM), jnp.float32),
            pltpu.SemaphoreType.DMA((NBUF,)),
            pltpu.SemaphoreType.DMA((NBUF,)),
        ],
        compiler_params=pltpu.CompilerParams(use_tc_tiling_on_sc=False),
    )
    def emb(idx_hbm, table_hbm, out_hbm, idx_v, rows_v, sem_g, sem_o):
        wid = lax.axis_index("s") * NUM_CORES + lax.axis_index("c")
        base = wid * rows_per_worker

        def fire_gathers(c, b):
            for j in range(NSUB):
                off = pl.multiple_of(c * CHUNK + j * SUB, SUB)
                pltpu.async_copy(
                    table_hbm.at[idx_v.at[pl.ds(off, SUB)]],
                    rows_v.at[b, pl.ds(j * SUB, SUB)],
                    sem_g.at[b],
                )

        def wait_gathers(c, b):
            for j in range(NSUB):
                off = pl.multiple_of(c * CHUNK + j * SUB, SUB)
                pltpu.make_async_copy(
                    table_hbm.at[idx_v.at[pl.ds(off, SUB)]],
                    rows_v.at[b, pl.ds(j * SUB, SUB)],
                    sem_g.at[b],
                ).wait()

        def fire_out(c, b):
            pltpu.async_copy(
                rows_v.at[b],
                out_hbm.at[pl.ds(base + c * CHUNK, CHUNK)],
                sem_o.at[b],
            )

        def wait_out(c, b):
            pltpu.make_async_copy(
                rows_v.at[b],
                out_hbm.at[pl.ds(base + c * CHUNK, CHUNK)],
                sem_o.at[b],
            ).wait()

        def scale_rows(b):
            @plsc.parallel_loop(0, CHUNK, unroll=8)
            def _(r):
                rows_v[b, r, pl.ds(0, 16)] = rows_v[b, r, pl.ds(0, 16)] * SCALE
                rows_v[b, r, pl.ds(16, 16)] = rows_v[b, r, pl.ds(16, 16)] * SCALE

        pltpu.sync_copy(idx_hbm.at[pl.ds(base, rows_per_worker)], idx_v)

        for b in range(NBUF):
            fire_gathers(b, b)

        def body(s, b, first, last):
            g = s * NBUF + b
            wait_gathers(g, b)
            scale_rows(b)
            fire_out(g, b)
            prev = (b - 1) % NBUF
            if not (first and b == 0) and not (last and b > 0):
                wait_out(g - 1, prev)
                fire_gathers(g + NBUF - 1, prev)

        for b in range(NBUF):
            body(0, b, True, num_super == 1)

        if num_super > 2:
            def super_body(s, carry):
                for b in range(NBUF):
                    body(s, b, False, False)
                return carry

            lax.fori_loop(1, num_super - 1, super_body, 0)

        if num_super > 1:
            for b in range(NBUF):
                body(num_super - 1, b, False, True)

        for b in range(NBUF):
            wait_out(num_chunks - NBUF + b, b)

    return emb


def kernel(x, table):
    idx = x.reshape(-1).astype(jnp.int32)
    out = _build(idx.shape[0])(idx, table)
    return out.reshape(x.shape + (EMBED_DIM,))


# speedup vs baseline: 1.7914x; 1.6214x over previous
import functools
import math

import jax
import jax.numpy as jnp
from jax import lax
from jax.experimental import pallas as pl
from jax.experimental.pallas import tpu as pltpu
from jax.experimental.pallas import tpu_sc as plsc

EMBED_DIM = 32
SCALE = math.sqrt(EMBED_DIM)

NUM_CORES = 2
NUM_SUBCORES = 16
NUM_WORKERS = NUM_CORES * NUM_SUBCORES

RW = 128
NWIN = 4
C0 = 10
NCHUNK = 5


@functools.lru_cache(maxsize=None)
def _build(num_rows: int, num_cols: int):
    assert num_rows == NUM_WORKERS * NWIN * RW
    assert num_cols == C0 * NCHUNK
    total = num_rows * num_cols
    mesh = plsc.VectorSubcoreMesh(core_axis_name="c", subcore_axis_name="s")

    @functools.partial(
        pl.kernel,
        mesh=mesh,
        out_type=jax.ShapeDtypeStruct((total * EMBED_DIM,), jnp.float32),
        scratch_types=[
            pltpu.VMEM((num_cols * RW,), jnp.int32),
            pltpu.VMEM((2, C0 * RW, EMBED_DIM), jnp.float32),
            pltpu.VMEM((C0 * EMBED_DIM * RW,), jnp.float32),
            pltpu.SemaphoreType.DMA,
            pltpu.SemaphoreType.DMA((2,)),
            pltpu.SemaphoreType.DMA,
        ],
        compiler_params=pltpu.CompilerParams(
            use_tc_tiling_on_sc=False, needs_layout_passes=False
        ),
    )
    def emb(idx_hbm, table_hbm, out_hbm, idx_v, rows_v, obuf_v, isem, gsem, osem):
        wid = lax.axis_index("s") * NUM_CORES + lax.axis_index("c")
        iota128 = lax.iota(jnp.int32, 16) * RW

        def fire_idx(r0):
            for c in range(num_cols):
                pltpu.async_copy(
                    idx_hbm.at[pl.ds(c * num_rows + r0, RW)],
                    idx_v.at[pl.ds(c * RW, RW)],
                    isem,
                )

        def wait_idx():
            pltpu.make_async_copy(
                idx_hbm.at[pl.ds(0, num_cols * RW)], idx_v, isem
            ).wait()

        def fire_gathers(cc, b):
            for c in range(C0):
                pltpu.async_copy(
                    table_hbm.at[idx_v.at[pl.ds((cc * C0 + c) * RW, RW)]],
                    rows_v.at[b, pl.ds(c * RW, RW)],
                    gsem.at[b],
                )

        def wait_gathers(b):
            pltpu.make_async_copy(
                table_hbm.at[pl.ds(0, C0 * RW)], rows_v.at[b], gsem.at[b]
            ).wait()

        def fire_out(cc, r0):
            for c in range(C0):
                for e in range(EMBED_DIM):
                    pltpu.async_copy(
                        obuf_v.at[pl.ds((c * EMBED_DIM + e) * RW, RW)],
                        out_hbm.at[
                            pl.ds(((cc * C0 + c) * EMBED_DIM + e) * num_rows + r0, RW)
                        ],
                        osem,
                    )

        def wait_out():
            pltpu.make_async_copy(
                out_hbm.at[pl.ds(0, C0 * EMBED_DIM * RW)], obuf_v, osem
            ).wait()

        def transpose_scale(b):
            for c in range(C0):
                @plsc.parallel_loop(0, RW, unroll=8)
                def _(r):
                    for h in range(EMBED_DIM // 16):
                        v = rows_v[b, c * RW + r, pl.ds(h * 16, 16)] * SCALE
                        tgt = iota128 + (c * EMBED_DIM * RW + h * 16 * RW + r)
                        plsc.store_scatter(obuf_v, [tgt], v)

        def window(k, carry):
            r0 = (wid * NWIN + k) * RW
            fire_idx(r0)
            wait_idx()
            fire_gathers(0, 0)
            for cc in range(NCHUNK):
                b = cc % 2
                if cc + 1 < NCHUNK:
                    fire_gathers(cc + 1, 1 - b)
                wait_gathers(b)
                if cc == 0:
                    @pl.when(k > 0)
                    def _():
                        wait_out()
                else:
                    wait_out()
                transpose_scale(b)
                fire_out(cc, r0)
            return carry

        lax.fori_loop(0, NWIN, window, 0)
        wait_out()

    return emb


def kernel(x, table):
    num_rows, num_cols = x.shape
    idx_t = jnp.swapaxes(x, 0, 1).reshape(-1).astype(jnp.int32)
    pout = _build(num_rows, num_cols)(idx_t, table)
    out = pout.reshape(num_cols, EMBED_DIM, num_rows)
    return jnp.transpose(out, (2, 0, 1))


# speedup vs baseline: 1.7996x; 1.0046x over previous
import functools
import math

import jax
import jax.numpy as jnp
from jax import lax
from jax.experimental import pallas as pl
from jax.experimental.pallas import tpu as pltpu
from jax.experimental.pallas import tpu_sc as plsc

EMBED_DIM = 32
SCALE = math.sqrt(EMBED_DIM)

NUM_CORES = 2
NUM_SUBCORES = 16
NUM_WORKERS = NUM_CORES * NUM_SUBCORES

RW = 128
NWIN = 4
C0 = 5
NCHUNK = 10


@functools.lru_cache(maxsize=None)
def _build_detile(num_rows: int, num_cols: int):
    n_tiles_c = (num_cols + 7) // 8
    n_win = num_rows // RW
    win_per_worker = n_win // NUM_WORKERS
    mesh = plsc.VectorSubcoreMesh(core_axis_name="c", subcore_axis_name="s")

    @functools.partial(
        pl.kernel,
        mesh=mesh,
        out_type=jax.ShapeDtypeStruct((num_rows * num_cols,), jnp.int32),
        scratch_types=[
            pltpu.VMEM((n_tiles_c * 8, RW), jnp.int32),
            pltpu.SemaphoreType.DMA,
            pltpu.SemaphoreType.DMA,
        ],
        compiler_params=pltpu.CompilerParams(
            use_tc_tiling_on_sc=True, needs_layout_passes=False
        ),
    )
    def detile(xt_hbm, out_hbm, stag, isem, osem):
        wid = lax.axis_index("s") * NUM_CORES + lax.axis_index("c")

        def win(k, carry):
            wdg = wid * win_per_worker + k
            r0 = wdg * RW
            for q in range(n_tiles_c):
                h = min(8, num_cols - q * 8)
                pltpu.async_copy(
                    xt_hbm.at[pl.ds(q * 8, h), pl.ds(r0, RW)],
                    stag.at[pl.ds(q * 8, h)],
                    isem,
                )
            for q in range(n_tiles_c):
                h = min(8, num_cols - q * 8)
                pltpu.make_async_copy(
                    xt_hbm.at[pl.ds(q * 8, h), pl.ds(r0, RW)],
                    stag.at[pl.ds(q * 8, h)],
                    isem,
                ).wait()
            for c in range(num_cols):
                pltpu.async_copy(
                    stag.at[c],
                    out_hbm.at[pl.ds((wdg * num_cols + c) * RW, RW)],
                    osem,
                )
            for c in range(num_cols):
                pltpu.make_async_copy(
                    stag.at[c],
                    out_hbm.at[pl.ds((wdg * num_cols + c) * RW, RW)],
                    osem,
                ).wait()
            return carry

        lax.fori_loop(0, win_per_worker, win, 0)

    return detile


@functools.lru_cache(maxsize=None)
def _build_lookup(num_rows: int, num_cols: int):
    assert num_rows == NUM_WORKERS * NWIN * RW
    assert num_cols == C0 * NCHUNK
    slab = num_cols * RW
    mesh = plsc.VectorSubcoreMesh(core_axis_name="c", subcore_axis_name="s")

    @functools.partial(
        pl.kernel,
        mesh=mesh,
        out_type=jax.ShapeDtypeStruct((num_cols * EMBED_DIM, num_rows), jnp.float32),
        scratch_types=[
            pltpu.VMEM((slab,), jnp.int32),
            pltpu.VMEM((2, C0 * RW, EMBED_DIM), jnp.float32),
            pltpu.VMEM((C0 * EMBED_DIM, RW), jnp.float32),
            pltpu.SemaphoreType.DMA,
            pltpu.SemaphoreType.DMA((2,)),
            pltpu.SemaphoreType.DMA,
        ],
        compiler_params=pltpu.CompilerParams(
            use_tc_tiling_on_sc=False, needs_layout_passes=False
        ),
    )
    def emb(idx_hbm, table_hbm, out_hbm, idx_v, rows_v, obuf_v, isem, gsem, osem):
        wid = lax.axis_index("s") * NUM_CORES + lax.axis_index("c")
        iota16 = lax.iota(jnp.int32, 16)

        def fire_gathers(cc, b):
            for c in range(C0):
                pltpu.async_copy(
                    table_hbm.at[idx_v.at[pl.ds((cc * C0 + c) * RW, RW)]],
                    rows_v.at[b, pl.ds(c * RW, RW)],
                    gsem.at[b],
                )

        def wait_gathers(b):
            pltpu.make_async_copy(
                table_hbm.at[pl.ds(0, C0 * RW)], rows_v.at[b], gsem.at[b]
            ).wait()

        def fire_out(cc, wdg):
            for c in range(C0):
                pltpu.async_copy(
                    obuf_v.at[pl.ds(c * EMBED_DIM, EMBED_DIM)],
                    out_hbm.at[
                        pl.ds((cc * C0 + c) * EMBED_DIM, EMBED_DIM),
                        pl.ds(wdg * RW, RW),
                    ],
                    osem,
                )

        def wait_out():
            pltpu.make_async_copy(
                out_hbm.at[pl.ds(0, C0 * EMBED_DIM), pl.ds(0, RW)], obuf_v, osem
            ).wait()

        def transpose_scale(b):
            for c in range(C0):
                row0 = iota16 + c * EMBED_DIM

                @plsc.parallel_loop(0, RW, unroll=8)
                def _(r):
                    colv = jnp.full((16,), r, jnp.int32)
                    for h in range(EMBED_DIM // 16):
                        v = rows_v[b, c * RW + r, pl.ds(h * 16, 16)] * SCALE
                        plsc.store_scatter(obuf_v, [row0 + h * 16, colv], v)

        def window(k, carry):
            wdg = wid * NWIN + k
            pltpu.async_copy(idx_hbm.at[pl.ds(wdg * slab, slab)], idx_v, isem)
            pltpu.make_async_copy(
                idx_hbm.at[pl.ds(0, slab)], idx_v, isem
            ).wait()
            fire_gathers(0, 0)
            for cc in range(NCHUNK):
                b = cc % 2
                if cc + 1 < NCHUNK:
                    fire_gathers(cc + 1, 1 - b)
                wait_gathers(b)
                if cc == 0:
                    @pl.when(k > 0)
                    def _():
                        wait_out()
                else:
                    wait_out()
                transpose_scale(b)
                fire_out(cc, wdg)
            return carry

        lax.fori_loop(0, NWIN, window, 0)
        wait_out()

    return emb


def kernel(x, table):
    num_rows, num_cols = x.shape
    xt = jnp.swapaxes(x, 0, 1).astype(jnp.int32)
    idx_lin = _build_detile(num_rows, num_cols)(xt)
    pout = _build_lookup(num_rows, num_cols)(idx_lin, table)
    out = pout.reshape(num_cols, EMBED_DIM, num_rows)
    return jnp.transpose(out, (2, 0, 1))
